# Initial kernel scaffold; baseline (speedup 1.0000x reference)
import jax
import jax.numpy as jnp
from jax.experimental import pallas as pl


def kernel(edge_index, x, W1, att_src1, att_dst1, b1, W2, att_src2, att_dst2, b2, mu_W, mu_b, lv_W, lv_b, dec_W, dec_b):
    raise NotImplementedError("write your pallas kernel here")


# speedup vs baseline: 26.0997x; 26.0997x over previous
import functools

import jax
import jax.numpy as jnp
from jax import lax
from jax.experimental import pallas as pl
from jax.experimental.pallas import tpu as pltpu
from jax.experimental.pallas import tpu_sc as plsc

N = 2048
E = 65536
F_IN = 256
NEURONS = 64
HEADS = 4
EMB = 16
HID = HEADS * NEURONS

NE = E + N
NSUB = 16
EPT = NE // NSUB
CHUNK = 128
NCHUNK = EPT // CHUNK
ROWS = 512
TRASH = ROWS * N
SLICE = ROWS * N // NSUB


def _count_body(src_hbm, dst_hbm, c_hbm, src_v, dst_v, idx_v, ones_v,
                zeros_v, shared):
    c = lax.axis_index("c")
    s = lax.axis_index("s")

    def _fill(i, _):
        zeros_v[pl.ds(i * 16, 16)] = jnp.zeros((16,), jnp.float32)
        return 0
    lax.fori_loop(0, 128, _fill, 0)

    def _fill1(i, _):
        ones_v[pl.ds(i * 16, 16)] = jnp.ones((16,), jnp.float32)
        return 0
    lax.fori_loop(0, CHUNK // 16, _fill1, 0)

    pltpu.sync_copy(src_hbm.at[pl.ds(s * EPT, EPT)], src_v)
    pltpu.sync_copy(dst_hbm.at[pl.ds(s * EPT, EPT)], dst_v)

    for rnd in range(2):
        base = (rnd * 2) * ROWS + c * ROWS

        def _zero(i, _):
            pltpu.sync_copy(zeros_v, shared.at[pl.ds(s * SLICE + i * 2048,
                                                     2048)])
            return 0
        lax.fori_loop(0, SLICE // 2048, _zero, 0)
        plsc.subcore_barrier()

        def _chunk(j, _):
            def _grp(k, _):
                off = j * CHUNK + k * 16
                d = dst_v[pl.ds(off, 16)]
                sv = src_v[pl.ds(off, 16)]
                rel = d - base
                inr = (rel >= 0) & (rel < ROWS)
                flat = rel * N + sv
                idx_v[j, pl.ds(k * 16, 16)] = jnp.where(inr, flat, TRASH)
                return 0
            lax.fori_loop(0, CHUNK // 16, _grp, 0)
            pltpu.sync_copy(ones_v, shared.at[idx_v.at[j]], add=True)
            return 0
        lax.fori_loop(0, NCHUNK, _chunk, 0)
        plsc.subcore_barrier()

        pltpu.sync_copy(shared.at[pl.ds(s * SLICE, SLICE)],
                        c_hbm.at[pl.ds(base * N + s * SLICE, SLICE)])
        plsc.subcore_barrier()


def _build_counts(src, dst):
    mesh = plsc.VectorSubcoreMesh(core_axis_name="c", subcore_axis_name="s")
    f = pl.kernel(
        _count_body,
        out_type=jax.ShapeDtypeStruct((N * N,), jnp.float32),
        mesh=mesh,
        scratch_types=[
            pltpu.VMEM((EPT,), jnp.int32),
            pltpu.VMEM((EPT,), jnp.int32),
            pltpu.VMEM((NCHUNK, CHUNK), jnp.int32),
            pltpu.VMEM((CHUNK,), jnp.float32),
            pltpu.VMEM((2048,), jnp.float32),
            pltpu.VMEM_SHARED((ROWS * N + 2048,), jnp.float32),
        ],
    )
    return f(src, dst)


BD = 256


def _proj1_body(x_ref, w1_ref, asrc_ref, adst_ref, xl_ref, asrcT_ref,
                adst_out_ref):
    xl = jnp.dot(x_ref[...], w1_ref[...], preferred_element_type=jnp.float32)
    xl_ref[...] = xl
    asrcT_ref[...] = lax.dot_general(
        asrc_ref[...], xl, (((1,), (1,)), ((), ())),
        preferred_element_type=jnp.float32)
    adst_out_ref[...] = jnp.dot(xl, adst_ref[...],
                                preferred_element_type=jnp.float32)


def _proj1(x, W1, AsrcM, AdstT):
    return pl.pallas_call(
        _proj1_body,
        grid=(N // BD,),
        in_specs=[
            pl.BlockSpec((BD, F_IN), lambda i: (i, 0)),
            pl.BlockSpec((F_IN, HID), lambda i: (0, 0)),
            pl.BlockSpec((HEADS, HID), lambda i: (0, 0)),
            pl.BlockSpec((HID, HEADS), lambda i: (0, 0)),
        ],
        out_specs=[
            pl.BlockSpec((BD, HID), lambda i: (i, 0)),
            pl.BlockSpec((HEADS, BD), lambda i: (0, i)),
            pl.BlockSpec((BD, HEADS), lambda i: (i, 0)),
        ],
        out_shape=[
            jax.ShapeDtypeStruct((N, HID), jnp.float32),
            jax.ShapeDtypeStruct((HEADS, N), jnp.float32),
            jax.ShapeDtypeStruct((N, HEADS), jnp.float32),
        ],
    )(x, W1, AsrcM, AdstT)


def _leaky(v):
    return jnp.where(v >= 0, v, 0.2 * v)


def _layer1_body(c_ref, asrcT_ref, adst_ref, xl_ref, w2_ref, a2s_ref,
                 a2d_ref, b1_ref, hl_ref, asrc2T_ref, adst2_ref):
    i = pl.program_id(0)
    m1 = _leaky(jnp.max(asrcT_ref[...], axis=1) +
                jnp.max(adst_ref[...], axis=0))
    adst_blk = adst_ref[pl.ds(i * BD, BD), :]
    cblk = c_ref[...]
    cols = []
    for h in range(HEADS):
        alpha = asrcT_ref[h:h + 1, :] + adst_blk[:, h:h + 1]
        p = cblk * jnp.exp(_leaky(alpha) - m1[h])
        den = jnp.sum(p, axis=1, keepdims=True)
        num = jnp.dot(p, xl_ref[:, h * NEURONS:(h + 1) * NEURONS],
                      preferred_element_type=jnp.float32)
        cols.append(jnp.maximum(
            num / den + b1_ref[0:1, h * NEURONS:(h + 1) * NEURONS], 0.0))
    hidden = jnp.concatenate(cols, axis=1)
    hl = jnp.dot(hidden, w2_ref[...], preferred_element_type=jnp.float32)
    hl_ref[...] = hl
    asrc2T_ref[...] = lax.dot_general(
        a2s_ref[...], hl, (((1,), (1,)), ((), ())),
        preferred_element_type=jnp.float32)
    adst2_ref[...] = jnp.dot(hl, a2d_ref[...],
                             preferred_element_type=jnp.float32)


def _layer1(C, a_srcT, a_dst, x_l, W2, att2s, att2dT, b1):
    return pl.pallas_call(
        _layer1_body,
        grid=(N // BD,),
        in_specs=[
            pl.BlockSpec((BD, N), lambda i: (i, 0)),
            pl.BlockSpec((HEADS, N), lambda i: (0, 0)),
            pl.BlockSpec((N, HEADS), lambda i: (0, 0)),
            pl.BlockSpec((N, HID), lambda i: (0, 0)),
            pl.BlockSpec((HID, EMB), lambda i: (0, 0)),
            pl.BlockSpec((1, EMB), lambda i: (0, 0)),
            pl.BlockSpec((EMB, 1), lambda i: (0, 0)),
            pl.BlockSpec((1, HID), lambda i: (0, 0)),
        ],
        out_specs=[
            pl.BlockSpec((BD, EMB), lambda i: (i, 0)),
            pl.BlockSpec((1, BD), lambda i: (0, i)),
            pl.BlockSpec((BD, 1), lambda i: (i, 0)),
        ],
        out_shape=[
            jax.ShapeDtypeStruct((N, EMB), jnp.float32),
            jax.ShapeDtypeStruct((1, N), jnp.float32),
            jax.ShapeDtypeStruct((N, 1), jnp.float32),
        ],
    )(C, a_srcT, a_dst, x_l, W2, att2s, att2dT, b1)


def _layer2_body(c_ref, hl_ref, asrc2T_ref, adst2_ref, b2_ref, emb_ref):
    i = pl.program_id(0)
    m2 = _leaky(jnp.max(asrc2T_ref[...]) + jnp.max(adst2_ref[...]))
    alpha = asrc2T_ref[...] + adst2_ref[pl.ds(i * BD, BD), :]
    p = c_ref[...] * jnp.exp(_leaky(alpha) - m2)
    den = jnp.sum(p, axis=1, keepdims=True)
    num = jnp.dot(p, hl_ref[...], preferred_element_type=jnp.float32)
    emb_ref[...] = num / den + b2_ref[...]


def _layer2(C, h_l, a_src2T, a_dst2, b2):
    return pl.pallas_call(
        _layer2_body,
        grid=(N // BD,),
        in_specs=[
            pl.BlockSpec((BD, N), lambda i: (i, 0)),
            pl.BlockSpec((N, EMB), lambda i: (0, 0)),
            pl.BlockSpec((1, N), lambda i: (0, 0)),
            pl.BlockSpec((N, 1), lambda i: (0, 0)),
            pl.BlockSpec((1, EMB), lambda i: (0, 0)),
        ],
        out_specs=pl.BlockSpec((BD, EMB), lambda i: (i, 0)),
        out_shape=jax.ShapeDtypeStruct((N, EMB), jnp.float32),
    )(C, h_l, a_src2T, a_dst2, b2)


def _vae_body(emb_ref, muw_ref, mub_ref, lvw_ref, lvb_ref, eps_ref, zm_ref):
    emb = emb_ref[...]
    mu = jnp.dot(emb, muw_ref[...], preferred_element_type=jnp.float32) \
        + mub_ref[...]
    lv = jnp.dot(emb, lvw_ref[...], preferred_element_type=jnp.float32) \
        + lvb_ref[...]
    z = mu + eps_ref[...] * jnp.exp(0.5 * lv)
    zm_ref[...] = jnp.mean(z, axis=0, keepdims=True)


def _vae(emb, mu_W, mu_b, lv_W, lv_b, eps):
    return pl.pallas_call(
        _vae_body,
        out_shape=jax.ShapeDtypeStruct((1, EMB), jnp.float32),
    )(emb, mu_W, mu_b, lv_W, lv_b, eps)


BR = 64


def _decode_body(zm_ref, w_ref, b_ref, out_ref):
    acc = b_ref[...]
    for k in range(EMB):
        acc = acc + zm_ref[0, k] * w_ref[k]
    out_ref[...] = 1.0 / (1.0 + jnp.exp(-acc))


def _decode(zm, dec_W3, dec_b2):
    return pl.pallas_call(
        _decode_body,
        grid=(N // BR,),
        in_specs=[
            pl.BlockSpec((1, EMB), lambda i: (0, 0)),
            pl.BlockSpec((EMB, BR, N), lambda i: (0, i, 0)),
            pl.BlockSpec((BR, N), lambda i: (i, 0)),
        ],
        out_specs=pl.BlockSpec((BR, N), lambda i: (i, 0)),
        out_shape=jax.ShapeDtypeStruct((N, N), jnp.float32),
    )(zm, dec_W3, dec_b2)


def kernel(edge_index, x, W1, att_src1, att_dst1, b1, W2, att_src2,
           att_dst2, b2, mu_W, mu_b, lv_W, lv_b, dec_W, dec_b):
    loops = jnp.arange(N, dtype=edge_index.dtype)
    src = jnp.concatenate([edge_index[0], loops])
    dst = jnp.concatenate([edge_index[1], loops])
    C = _build_counts(src, dst).reshape(N, N)

    AsrcM = (jnp.eye(HEADS, dtype=jnp.float32)[:, :, None]
             * att_src1[0][:, None, :]).reshape(HEADS, HID)
    AdstM = (jnp.eye(HEADS, dtype=jnp.float32)[:, :, None]
             * att_dst1[0][:, None, :]).reshape(HEADS, HID)
    x_l, a_srcT, a_dst = _proj1(x, W1, AsrcM, AdstM.T)

    h_l, a_src2T, a_dst2 = _layer1(
        C, a_srcT, a_dst, x_l, W2,
        att_src2.reshape(1, EMB), att_dst2.reshape(1, EMB).T,
        b1.reshape(1, HID))

    emb = _layer2(C, h_l, a_src2T, a_dst2, b2.reshape(1, EMB))

    eps = jax.random.normal(jax.random.key(42), (N, EMB), jnp.float32)
    zm = _vae(emb, mu_W, mu_b.reshape(1, EMB), lv_W, lv_b.reshape(1, EMB),
              eps)

    return _decode(zm, dec_W.reshape(EMB, N, N), dec_b.reshape(N, N))


# speedup vs baseline: 37.5626x; 1.4392x over previous
import functools

import jax
import jax.numpy as jnp
from jax import lax
from jax.experimental import pallas as pl
from jax.experimental.pallas import tpu as pltpu
from jax.experimental.pallas import tpu_sc as plsc

N = 2048
E = 65536
F_IN = 256
NEURONS = 64
HEADS = 4
EMB = 16
HID = HEADS * NEURONS

NE = E + N
NSUB = 16
EPT = NE // NSUB
CHUNK = 128
NCHUNK = EPT // CHUNK
ROWS = 512
TRASH = ROWS * N
SLICE = ROWS * N // NSUB
ZBUF = 16384


def _count_body(src_hbm, dst_hbm, c_hbm, src_v, dst_v, idx_v, ones_v,
                zeros_v, shared, sem):
    c = lax.axis_index("c")
    s = lax.axis_index("s")

    def _fill(i, _):
        zeros_v[pl.ds(i * 16, 16)] = jnp.zeros((16,), jnp.float32)
        return 0
    lax.fori_loop(0, ZBUF // 16, _fill, 0)

    def _fill1(i, _):
        ones_v[i // 8, pl.ds((i % 8) * 16, 16)] = jnp.ones((16,), jnp.float32)
        return 0
    lax.fori_loop(0, NCHUNK * 8, _fill1, 0)

    pltpu.sync_copy(src_hbm.at[pl.ds(s * EPT, EPT)], src_v)
    pltpu.sync_copy(dst_hbm.at[pl.ds(s * EPT, EPT)], dst_v)

    for rnd in range(2):
        base = (rnd * 2) * ROWS + c * ROWS

        zcopies = [
            pltpu.async_copy(
                zeros_v, shared.at[pl.ds(s * SLICE + i * ZBUF, ZBUF)], sem)
            for i in range(SLICE // ZBUF)
        ]
        def _grp(k, _):
            off = k * 16
            d = dst_v[pl.ds(off, 16)]
            sv = src_v[pl.ds(off, 16)]
            rel = d - base
            inr = (rel >= 0) & (rel < ROWS)
            flat = rel * N + sv
            idx_v[off // CHUNK, pl.ds(off % CHUNK, 16)] = \
                jnp.where(inr, flat, TRASH)
            return 0
        lax.fori_loop(0, EPT // 16, _grp, 0)
        for zc in zcopies:
            zc.wait()
        plsc.subcore_barrier()

        scopies = [
            pltpu.async_copy(ones_v.at[j], shared.at[idx_v.at[j]], sem,
                             add=True)
            for j in range(NCHUNK)
        ]
        for sc_ in scopies:
            sc_.wait()
        plsc.subcore_barrier()

        pltpu.sync_copy(shared.at[pl.ds(s * SLICE, SLICE)],
                        c_hbm.at[pl.ds(base * N + s * SLICE, SLICE)])
        plsc.subcore_barrier()


def _build_counts(src, dst):
    mesh = plsc.VectorSubcoreMesh(core_axis_name="c", subcore_axis_name="s")
    f = pl.kernel(
        _count_body,
        out_type=jax.ShapeDtypeStruct((N * N,), jnp.float32),
        mesh=mesh,
        scratch_types=[
            pltpu.VMEM((EPT,), jnp.int32),
            pltpu.VMEM((EPT,), jnp.int32),
            pltpu.VMEM((NCHUNK, CHUNK), jnp.int32),
            pltpu.VMEM((NCHUNK, CHUNK), jnp.float32),
            pltpu.VMEM((ZBUF,), jnp.float32),
            pltpu.VMEM_SHARED((ROWS * N + 2048,), jnp.float32),
            pltpu.SemaphoreType.DMA,
        ],
    )
    return f(src, dst)


BD = 256


def _proj1_body(x_ref, w1_ref, asrc_ref, adst_ref, xl_ref, asrcT_ref,
                adst_out_ref):
    xl = jnp.dot(x_ref[...], w1_ref[...], preferred_element_type=jnp.float32)
    xl_ref[...] = xl
    asrcT_ref[...] = lax.dot_general(
        asrc_ref[...], xl, (((1,), (1,)), ((), ())),
        preferred_element_type=jnp.float32)
    adst_out_ref[...] = jnp.dot(xl, adst_ref[...],
                                preferred_element_type=jnp.float32)


def _proj1(x, W1, AsrcM, AdstT):
    return pl.pallas_call(
        _proj1_body,
        grid=(N // BD,),
        in_specs=[
            pl.BlockSpec((BD, F_IN), lambda i: (i, 0)),
            pl.BlockSpec((F_IN, HID), lambda i: (0, 0)),
            pl.BlockSpec((HEADS, HID), lambda i: (0, 0)),
            pl.BlockSpec((HID, HEADS), lambda i: (0, 0)),
        ],
        out_specs=[
            pl.BlockSpec((BD, HID), lambda i: (i, 0)),
            pl.BlockSpec((HEADS, BD), lambda i: (0, i)),
            pl.BlockSpec((BD, HEADS), lambda i: (i, 0)),
        ],
        out_shape=[
            jax.ShapeDtypeStruct((N, HID), jnp.float32),
            jax.ShapeDtypeStruct((HEADS, N), jnp.float32),
            jax.ShapeDtypeStruct((N, HEADS), jnp.float32),
        ],
    )(x, W1, AsrcM, AdstT)


def _leaky(v):
    return jnp.where(v >= 0, v, 0.2 * v)


def _layer1_body(c_ref, asrcT_ref, adst_ref, xl_ref, w2_ref, a2s_ref,
                 a2d_ref, b1_ref, hl_ref, asrc2T_ref, adst2_ref):
    i = pl.program_id(0)
    m1 = _leaky(jnp.max(asrcT_ref[...], axis=1) +
                jnp.max(adst_ref[...], axis=0))
    adst_blk = adst_ref[pl.ds(i * BD, BD), :]
    cblk = c_ref[...]
    cols = []
    for h in range(HEADS):
        alpha = asrcT_ref[h:h + 1, :] + adst_blk[:, h:h + 1]
        p = cblk * jnp.exp(_leaky(alpha) - m1[h])
        den = jnp.sum(p, axis=1, keepdims=True)
        num = jnp.dot(p, xl_ref[:, h * NEURONS:(h + 1) * NEURONS],
                      preferred_element_type=jnp.float32)
        cols.append(jnp.maximum(
            num / den + b1_ref[0:1, h * NEURONS:(h + 1) * NEURONS], 0.0))
    hidden = jnp.concatenate(cols, axis=1)
    hl = jnp.dot(hidden, w2_ref[...], preferred_element_type=jnp.float32)
    hl_ref[...] = hl
    asrc2T_ref[...] = lax.dot_general(
        a2s_ref[...], hl, (((1,), (1,)), ((), ())),
        preferred_element_type=jnp.float32)
    adst2_ref[...] = jnp.dot(hl, a2d_ref[...],
                             preferred_element_type=jnp.float32)


def _layer1(C, a_srcT, a_dst, x_l, W2, att2s, att2dT, b1):
    return pl.pallas_call(
        _layer1_body,
        grid=(N // BD,),
        in_specs=[
            pl.BlockSpec((BD, N), lambda i: (i, 0)),
            pl.BlockSpec((HEADS, N), lambda i: (0, 0)),
            pl.BlockSpec((N, HEADS), lambda i: (0, 0)),
            pl.BlockSpec((N, HID), lambda i: (0, 0)),
            pl.BlockSpec((HID, EMB), lambda i: (0, 0)),
            pl.BlockSpec((1, EMB), lambda i: (0, 0)),
            pl.BlockSpec((EMB, 1), lambda i: (0, 0)),
            pl.BlockSpec((1, HID), lambda i: (0, 0)),
        ],
        out_specs=[
            pl.BlockSpec((BD, EMB), lambda i: (i, 0)),
            pl.BlockSpec((1, BD), lambda i: (0, i)),
            pl.BlockSpec((BD, 1), lambda i: (i, 0)),
        ],
        out_shape=[
            jax.ShapeDtypeStruct((N, EMB), jnp.float32),
            jax.ShapeDtypeStruct((1, N), jnp.float32),
            jax.ShapeDtypeStruct((N, 1), jnp.float32),
        ],
    )(C, a_srcT, a_dst, x_l, W2, att2s, att2dT, b1)


def _layer2_body(c_ref, hl_ref, asrc2T_ref, adst2_ref, b2_ref, emb_ref):
    i = pl.program_id(0)
    m2 = _leaky(jnp.max(asrc2T_ref[...]) + jnp.max(adst2_ref[...]))
    alpha = asrc2T_ref[...] + adst2_ref[pl.ds(i * BD, BD), :]
    p = c_ref[...] * jnp.exp(_leaky(alpha) - m2)
    den = jnp.sum(p, axis=1, keepdims=True)
    num = jnp.dot(p, hl_ref[...], preferred_element_type=jnp.float32)
    emb_ref[...] = num / den + b2_ref[...]


def _layer2(C, h_l, a_src2T, a_dst2, b2):
    return pl.pallas_call(
        _layer2_body,
        grid=(N // BD,),
        in_specs=[
            pl.BlockSpec((BD, N), lambda i: (i, 0)),
            pl.BlockSpec((N, EMB), lambda i: (0, 0)),
            pl.BlockSpec((1, N), lambda i: (0, 0)),
            pl.BlockSpec((N, 1), lambda i: (0, 0)),
            pl.BlockSpec((1, EMB), lambda i: (0, 0)),
        ],
        out_specs=pl.BlockSpec((BD, EMB), lambda i: (i, 0)),
        out_shape=jax.ShapeDtypeStruct((N, EMB), jnp.float32),
    )(C, h_l, a_src2T, a_dst2, b2)


def _vae_body(emb_ref, muw_ref, mub_ref, lvw_ref, lvb_ref, eps_ref, zm_ref):
    emb = emb_ref[...]
    mu = jnp.dot(emb, muw_ref[...], preferred_element_type=jnp.float32) \
        + mub_ref[...]
    lv = jnp.dot(emb, lvw_ref[...], preferred_element_type=jnp.float32) \
        + lvb_ref[...]
    z = mu + eps_ref[...] * jnp.exp(0.5 * lv)
    zm_ref[...] = jnp.mean(z, axis=0, keepdims=True)


def _vae(emb, mu_W, mu_b, lv_W, lv_b, eps):
    return pl.pallas_call(
        _vae_body,
        out_shape=jax.ShapeDtypeStruct((1, EMB), jnp.float32),
    )(emb, mu_W, mu_b, lv_W, lv_b, eps)


BR = 64


def _decode_body(zm_ref, w_ref, b_ref, out_ref):
    y = jnp.dot(zm_ref[...], w_ref[...], preferred_element_type=jnp.float32)
    y2 = y.reshape(BR, N) + b_ref[...]
    out_ref[...] = 1.0 / (1.0 + jnp.exp(-y2))


def _decode(zm, dec_W, dec_b2):
    return pl.pallas_call(
        _decode_body,
        grid=(N // BR,),
        in_specs=[
            pl.BlockSpec((1, EMB), lambda i: (0, 0)),
            pl.BlockSpec((EMB, BR * N), lambda i: (0, i)),
            pl.BlockSpec((BR, N), lambda i: (i, 0)),
        ],
        out_specs=pl.BlockSpec((BR, N), lambda i: (i, 0)),
        out_shape=jax.ShapeDtypeStruct((N, N), jnp.float32),
    )(zm, dec_W, dec_b2)


def kernel(edge_index, x, W1, att_src1, att_dst1, b1, W2, att_src2,
           att_dst2, b2, mu_W, mu_b, lv_W, lv_b, dec_W, dec_b):
    loops = jnp.arange(N, dtype=edge_index.dtype)
    src = jnp.concatenate([edge_index[0], loops])
    dst = jnp.concatenate([edge_index[1], loops])
    C = _build_counts(src, dst).reshape(N, N)

    AsrcM = (jnp.eye(HEADS, dtype=jnp.float32)[:, :, None]
             * att_src1[0][:, None, :]).reshape(HEADS, HID)
    AdstM = (jnp.eye(HEADS, dtype=jnp.float32)[:, :, None]
             * att_dst1[0][:, None, :]).reshape(HEADS, HID)
    x_l, a_srcT, a_dst = _proj1(x, W1, AsrcM, AdstM.T)

    h_l, a_src2T, a_dst2 = _layer1(
        C, a_srcT, a_dst, x_l, W2,
        att_src2.reshape(1, EMB), att_dst2.reshape(1, EMB).T,
        b1.reshape(1, HID))

    emb = _layer2(C, h_l, a_src2T, a_dst2, b2.reshape(1, EMB))

    eps = jax.random.normal(jax.random.key(42), (N, EMB), jnp.float32)
    zm = _vae(emb, mu_W, mu_b.reshape(1, EMB), lv_W, lv_b.reshape(1, EMB),
              eps)

    return _decode(zm, dec_W, dec_b.reshape(N, N))


# speedup vs baseline: 54.8068x; 1.4591x over previous
import functools

import jax
import jax.numpy as jnp
from jax import lax
from jax.experimental import pallas as pl
from jax.experimental.pallas import tpu as pltpu
from jax.experimental.pallas import tpu_sc as plsc

N = 2048
E = 65536
F_IN = 256
NEURONS = 64
HEADS = 4
EMB = 16
HID = HEADS * NEURONS

NE = E + N
NSUB = 16
EPT = NE // NSUB
CHUNK = 128
NCHUNK = EPT // CHUNK
ROWS = 512
TRASH = ROWS * N
SLICE = ROWS * N // NSUB
ZBUF = 16384
NCMAX = EPT // CHUNK + 1


def _count_body(src_hbm, dst_hbm, c_hbm, src_v, dst_v, idx_v, ones_v,
                zeros_v, shared, sem):
    c = lax.axis_index("c")
    s = lax.axis_index("s")

    def _fill(i, _):
        zeros_v[pl.ds(i * 16, 16)] = jnp.zeros((16,), jnp.float32)
        return 0
    lax.fori_loop(0, ZBUF // 16, _fill, 0)

    def _fill1(i, _):
        ones_v[pl.ds(i * 16, 16)] = jnp.ones((16,), jnp.float32)
        return 0
    lax.fori_loop(0, CHUNK // 16, _fill1, 0)

    pltpu.sync_copy(src_hbm.at[pl.ds(s * EPT, EPT)], src_v)
    pltpu.sync_copy(dst_hbm.at[pl.ds(s * EPT, EPT)], dst_v)

    for rnd in range(2):
        base = (rnd * 2) * ROWS + c * ROWS

        zcopies = [
            pltpu.async_copy(
                zeros_v, shared.at[pl.ds(s * SLICE + i * ZBUF, ZBUF)], sem)
            for i in range(SLICE // ZBUF)
        ]
        def _grp(k, _):
            off = k * 16
            d = dst_v[pl.ds(off, 16)]
            sv = src_v[pl.ds(off, 16)]
            rel = d - base
            inr = (rel >= 0) & (rel < ROWS)
            flat = rel * N + sv
            idx_v[off // CHUNK, pl.ds(off % CHUNK, 16)] = \
                jnp.where(inr, flat, TRASH + (sv & 1023))
            return 0
        lax.fori_loop(0, EPT // 16, _grp, 0)
        for zc in zcopies:
            zc.wait()
        plsc.subcore_barrier()

        scopies = [
            pltpu.async_copy(ones_v, shared.at[idx_v.at[j]], sem,
                             add=True)
            for j in range(NCHUNK)
        ]
        for sc_ in scopies:
            sc_.wait()
        plsc.subcore_barrier()

        pltpu.sync_copy(shared.at[pl.ds(s * SLICE, SLICE)],
                        c_hbm.at[pl.ds(base * N + s * SLICE, SLICE)])
        plsc.subcore_barrier()


def _build_counts(src, dst):
    mesh = plsc.VectorSubcoreMesh(core_axis_name="c", subcore_axis_name="s")
    f = pl.kernel(
        _count_body,
        out_type=jax.ShapeDtypeStruct((N * N,), jnp.float32),
        mesh=mesh,
        scratch_types=[
            pltpu.VMEM((EPT,), jnp.int32),
            pltpu.VMEM((EPT,), jnp.int32),
            pltpu.VMEM((NCHUNK, CHUNK), jnp.int32),
            pltpu.VMEM((CHUNK,), jnp.float32),
            pltpu.VMEM((ZBUF,), jnp.float32),
            pltpu.VMEM_SHARED((ROWS * N + 2048,), jnp.float32),
            pltpu.SemaphoreType.DMA,
        ],
    )
    return f(src, dst)


BD = 256


def _proj1_body(x_ref, w1_ref, asrc_ref, adst_ref, xl_ref, asrcT_ref,
                adst_out_ref):
    xl = jnp.dot(x_ref[...], w1_ref[...], preferred_element_type=jnp.float32)
    xl_ref[...] = xl
    asrcT_ref[...] = lax.dot_general(
        asrc_ref[...], xl, (((1,), (1,)), ((), ())),
        preferred_element_type=jnp.float32)
    adst_out_ref[...] = jnp.dot(xl, adst_ref[...],
                                preferred_element_type=jnp.float32)


def _proj1(x, W1, AsrcM, AdstT):
    return pl.pallas_call(
        _proj1_body,
        grid=(N // BD,),
        in_specs=[
            pl.BlockSpec((BD, F_IN), lambda i: (i, 0)),
            pl.BlockSpec((F_IN, HID), lambda i: (0, 0)),
            pl.BlockSpec((HEADS, HID), lambda i: (0, 0)),
            pl.BlockSpec((HID, HEADS), lambda i: (0, 0)),
        ],
        out_specs=[
            pl.BlockSpec((BD, HID), lambda i: (i, 0)),
            pl.BlockSpec((HEADS, BD), lambda i: (0, i)),
            pl.BlockSpec((BD, HEADS), lambda i: (i, 0)),
        ],
        out_shape=[
            jax.ShapeDtypeStruct((N, HID), jnp.float32),
            jax.ShapeDtypeStruct((HEADS, N), jnp.float32),
            jax.ShapeDtypeStruct((N, HEADS), jnp.float32),
        ],
    )(x, W1, AsrcM, AdstT)


def _leaky(v):
    return jnp.where(v >= 0, v, 0.2 * v)


def _layer1_body(c_ref, asrcT_ref, adst_ref, xl_ref, w2_ref, a2s_ref,
                 a2d_ref, b1_ref, hl_ref, asrc2T_ref, adst2_ref):
    i = pl.program_id(0)
    m1 = _leaky(jnp.max(asrcT_ref[...], axis=1) +
                jnp.max(adst_ref[...], axis=0))
    adst_blk = adst_ref[pl.ds(i * BD, BD), :]
    cblk = c_ref[...].astype(jnp.float32)
    cols = []
    for h in range(HEADS):
        alpha = asrcT_ref[h:h + 1, :] + adst_blk[:, h:h + 1]
        p = cblk * jnp.exp(_leaky(alpha) - m1[h])
        den = jnp.sum(p, axis=1, keepdims=True)
        num = jnp.dot(p, xl_ref[:, h * NEURONS:(h + 1) * NEURONS],
                      preferred_element_type=jnp.float32)
        cols.append(jnp.maximum(
            num / den + b1_ref[0:1, h * NEURONS:(h + 1) * NEURONS], 0.0))
    hidden = jnp.concatenate(cols, axis=1)
    hl = jnp.dot(hidden, w2_ref[...], preferred_element_type=jnp.float32)
    hl_ref[...] = hl
    asrc2T_ref[...] = lax.dot_general(
        a2s_ref[...], hl, (((1,), (1,)), ((), ())),
        preferred_element_type=jnp.float32)
    adst2_ref[...] = jnp.dot(hl, a2d_ref[...],
                             preferred_element_type=jnp.float32)


def _layer1(C, a_srcT, a_dst, x_l, W2, att2s, att2dT, b1):
    return pl.pallas_call(
        _layer1_body,
        grid=(N // BD,),
        in_specs=[
            pl.BlockSpec((BD, N), lambda i: (i, 0)),
            pl.BlockSpec((HEADS, N), lambda i: (0, 0)),
            pl.BlockSpec((N, HEADS), lambda i: (0, 0)),
            pl.BlockSpec((N, HID), lambda i: (0, 0)),
            pl.BlockSpec((HID, EMB), lambda i: (0, 0)),
            pl.BlockSpec((1, EMB), lambda i: (0, 0)),
            pl.BlockSpec((EMB, 1), lambda i: (0, 0)),
            pl.BlockSpec((1, HID), lambda i: (0, 0)),
        ],
        out_specs=[
            pl.BlockSpec((BD, EMB), lambda i: (i, 0)),
            pl.BlockSpec((1, BD), lambda i: (0, i)),
            pl.BlockSpec((BD, 1), lambda i: (i, 0)),
        ],
        out_shape=[
            jax.ShapeDtypeStruct((N, EMB), jnp.float32),
            jax.ShapeDtypeStruct((1, N), jnp.float32),
            jax.ShapeDtypeStruct((N, 1), jnp.float32),
        ],
    )(C, a_srcT, a_dst, x_l, W2, att2s, att2dT, b1)


def _layer2_body(c_ref, hl_ref, asrc2T_ref, adst2_ref, b2_ref, emb_ref):
    i = pl.program_id(0)
    m2 = _leaky(jnp.max(asrc2T_ref[...]) + jnp.max(adst2_ref[...]))
    alpha = asrc2T_ref[...] + adst2_ref[pl.ds(i * BD, BD), :]
    p = c_ref[...].astype(jnp.float32) * jnp.exp(_leaky(alpha) - m2)
    den = jnp.sum(p, axis=1, keepdims=True)
    num = jnp.dot(p, hl_ref[...], preferred_element_type=jnp.float32)
    emb_ref[...] = num / den + b2_ref[...]


def _layer2(C, h_l, a_src2T, a_dst2, b2):
    return pl.pallas_call(
        _layer2_body,
        grid=(N // BD,),
        in_specs=[
            pl.BlockSpec((BD, N), lambda i: (i, 0)),
            pl.BlockSpec((N, EMB), lambda i: (0, 0)),
            pl.BlockSpec((1, N), lambda i: (0, 0)),
            pl.BlockSpec((N, 1), lambda i: (0, 0)),
            pl.BlockSpec((1, EMB), lambda i: (0, 0)),
        ],
        out_specs=pl.BlockSpec((BD, EMB), lambda i: (i, 0)),
        out_shape=jax.ShapeDtypeStruct((N, EMB), jnp.float32),
    )(C, h_l, a_src2T, a_dst2, b2)


def _vae_body(emb_ref, muw_ref, mub_ref, lvw_ref, lvb_ref, eps_ref, zm_ref):
    emb = emb_ref[...]
    mu = jnp.dot(emb, muw_ref[...], preferred_element_type=jnp.float32) \
        + mub_ref[...]
    lv = jnp.dot(emb, lvw_ref[...], preferred_element_type=jnp.float32) \
        + lvb_ref[...]
    z = mu + eps_ref[...] * jnp.exp(0.5 * lv)
    zm_ref[...] = jnp.mean(z, axis=0, keepdims=True)


def _vae(emb, mu_W, mu_b, lv_W, lv_b, eps):
    return pl.pallas_call(
        _vae_body,
        out_shape=jax.ShapeDtypeStruct((1, EMB), jnp.float32),
    )(emb, mu_W, mu_b, lv_W, lv_b, eps)


BR = 64


def _decode_body(zm_ref, w_ref, b_ref, out_ref):
    y = jnp.dot(zm_ref[...], w_ref[...], preferred_element_type=jnp.float32)
    y2 = y.reshape(BR, N) + b_ref[...]
    out_ref[...] = 1.0 / (1.0 + jnp.exp(-y2))


def _decode(zm, dec_W, dec_b2):
    return pl.pallas_call(
        _decode_body,
        grid=(N // BR,),
        in_specs=[
            pl.BlockSpec((1, EMB), lambda i: (0, 0)),
            pl.BlockSpec((EMB, BR * N), lambda i: (0, i)),
            pl.BlockSpec((BR, N), lambda i: (i, 0)),
        ],
        out_specs=pl.BlockSpec((BR, N), lambda i: (i, 0)),
        out_shape=jax.ShapeDtypeStruct((N, N), jnp.float32),
    )(zm, dec_W, dec_b2)


def kernel(edge_index, x, W1, att_src1, att_dst1, b1, W2, att_src2,
           att_dst2, b2, mu_W, mu_b, lv_W, lv_b, dec_W, dec_b):
    loops = jnp.arange(N, dtype=edge_index.dtype)
    src = jnp.concatenate([edge_index[0], loops])
    dst = jnp.concatenate([edge_index[1], loops])
    C = _build_counts(src, dst).reshape(N, N)

    AsrcM = (jnp.eye(HEADS, dtype=jnp.float32)[:, :, None]
             * att_src1[0][:, None, :]).reshape(HEADS, HID)
    AdstM = (jnp.eye(HEADS, dtype=jnp.float32)[:, :, None]
             * att_dst1[0][:, None, :]).reshape(HEADS, HID)
    x_l, a_srcT, a_dst = _proj1(x, W1, AsrcM, AdstM.T)

    h_l, a_src2T, a_dst2 = _layer1(
        C, a_srcT, a_dst, x_l, W2,
        att_src2.reshape(1, EMB), att_dst2.reshape(1, EMB).T,
        b1.reshape(1, HID))

    emb = _layer2(C, h_l, a_src2T, a_dst2, b2.reshape(1, EMB))

    eps = jax.random.normal(jax.random.key(42), (N, EMB), jnp.float32)
    zm = _vae(emb, mu_W, mu_b.reshape(1, EMB), lv_W, lv_b.reshape(1, EMB),
              eps)

    return _decode(zm, dec_W, dec_b.reshape(N, N))
